# Initial kernel scaffold; baseline (speedup 1.0000x reference)
import jax
import jax.numpy as jnp
from jax.experimental import pallas as pl


def kernel(x_nchw, w1, b1, g1, beta1, w2, b2, w3, b3, g3, beta3, wf, bf):
    raise NotImplementedError("write your pallas kernel here")


# speedup vs baseline: 2.2456x; 2.2456x over previous
import functools

import jax
import jax.numpy as jnp
from jax import lax
from jax.experimental import pallas as pl
from jax.experimental.pallas import tpu as pltpu

EPS = 1e-5
VMEM_LIMIT = 32 * 1024 * 1024


def _cparams():
    return pltpu.CompilerParams(dimension_semantics=("parallel",),
                                vmem_limit_bytes=VMEM_LIMIT)


def _tap_offsets(wp):
    return tuple((dy - 1) * wp + (dx - 1) for dy in range(3) for dx in range(3))


def _conv_weight_mat(w_oihw):
    o, i, kh, kw = w_oihw.shape
    return jnp.transpose(w_oihw, (0, 2, 3, 1)).reshape(o, kh * kw * i)


def _interior_mask(h, w):
    m = jnp.zeros((h + 2, w + 2), jnp.float32)
    m = m.at[1:h + 1, 1:w + 1].set(1.0)
    return m.reshape(1, (h + 2) * (w + 2))


def _build_patches(x, p_ref, offsets):
    cin, L = x.shape
    for t, off in enumerate(offsets):
        r = t * cin
        if off >= 0:
            p_ref[r:r + cin, 0:L - off] = x[:, off:L]
        else:
            p_ref[r:r + cin, -off:L] = x[:, 0:L + off]


def _stats1_kernel(x_ref, w_ref, b_ref, m_ref, st_ref, p_ref, *, offsets):
    _build_patches(x_ref[0], p_ref, offsets)
    y = jnp.dot(w_ref[...], p_ref[...], preferred_element_type=jnp.float32)
    y = y + b_ref[...]
    ym = jnp.where(m_ref[...] > 0.5, y, 0.0)
    st_ref[0, :, 0:1] = jnp.sum(ym, axis=1, keepdims=True)
    st_ref[0, :, 1:2] = jnp.sum(ym * ym, axis=1, keepdims=True)


def _main_kernel(x_ref, w1_ref, b1_ref, w2_ref, b2_ref, w3_ref, b3_ref,
                 m1_ref, m2_ref, s_ref, y3_ref, st_ref,
                 p1_ref, f_ref, d_ref, q_ref, p2_ref, p3_ref,
                 *, off1, off2, h, w):
    hd, wd = h // 2, w // 2
    c1 = w1_ref.shape[0]
    R = c1 * (h + 2)
    _build_patches(x_ref[0], p1_ref, off1)
    y1 = jnp.dot(w1_ref[...], p1_ref[...], preferred_element_type=jnp.float32)
    a1 = jnp.where(m1_ref[...] > 0.5,
                   jnp.maximum(y1 + b1_ref[...], 0.0), 0.0)
    f_ref[...] = a1.reshape(c1, h + 2, w + 2)
    A = f_ref[...].reshape(R, w + 2)
    hm = jnp.maximum(A[:, 0:w + 1], A[:, 1:w + 2])
    dec = jnp.dot(hm, s_ref[...], preferred_element_type=jnp.float32)
    d_ref[0:R, 0:wd] = dec
    v0 = d_ref[1:R + 1:2, 0:wd]
    v1 = d_ref[2:R + 2:2, 0:wd]
    pooled = jnp.maximum(v0, v1).astype(jnp.bfloat16)
    q_ref[...] = jnp.zeros(q_ref.shape, q_ref.dtype)
    for c in range(c1):
        q_ref[c, 1:hd + 1, 1:wd + 1] = pooled[c * (hd + 1):c * (hd + 1) + hd, :]
    p1 = q_ref[...].reshape(c1, (hd + 2) * (wd + 2))

    _build_patches(p1, p2_ref, off2)
    y2 = jnp.dot(w2_ref[...], p2_ref[...], preferred_element_type=jnp.float32)
    interior = m2_ref[...] > 0.5
    a2 = jnp.where(interior, jnp.maximum(y2 + b2_ref[...], 0.0),
                   0.0).astype(jnp.bfloat16)

    _build_patches(a2, p3_ref, off2)
    y3 = jnp.dot(w3_ref[...], p3_ref[...], preferred_element_type=jnp.float32)
    y3 = jnp.where(interior, y3 + b3_ref[...], 0.0)
    st_ref[0, :, 0:1] = jnp.sum(y3, axis=1, keepdims=True)
    st_ref[0, :, 1:2] = jnp.sum(y3 * y3, axis=1, keepdims=True)
    y3_ref[0] = y3.astype(jnp.bfloat16)


def _head_kernel(y_ref, sc_ref, sh_ref, wf_ref, o_ref):
    y = y_ref[0].astype(jnp.float32)
    a = jnp.maximum(y * sc_ref[...] + sh_ref[...], 0.0)
    o_ref[0] = jnp.sum(a * wf_ref[...], axis=1, keepdims=True)


def _finalize_bn(stats, gamma, beta, count):
    tot = jnp.sum(stats, axis=0)
    mean = tot[:, 0] / count
    var = jnp.maximum(tot[:, 1] / count - mean * mean, 0.0)
    scale = gamma * lax.rsqrt(var + EPS)
    shift = beta - mean * scale
    return scale, shift


def kernel(x_nchw, w1, b1, g1, beta1, w2, b2, w3, b3, g3, beta3, wf, bf):
    n, cin, h, w = x_nchw.shape
    c1, c2, c3 = w1.shape[0], w2.shape[0], w3.shape[0]
    hd, wd = h // 2, w // 2
    L1 = (h + 2) * (w + 2)
    L2 = (hd + 2) * (wd + 2)
    off1 = _tap_offsets(w + 2)
    off2 = _tap_offsets(wd + 2)

    xpf = jnp.pad(x_nchw, ((0, 0), (0, 0), (1, 1), (1, 1))) \
             .reshape(n, cin, L1).astype(jnp.bfloat16)
    mask1 = _interior_mask(h, w)
    mask2 = _interior_mask(hd, wd)

    w1m = _conv_weight_mat(w1)

    st1 = pl.pallas_call(
        functools.partial(_stats1_kernel, offsets=off1),
        grid=(n,),
        in_specs=[
            pl.BlockSpec((1, cin, L1), lambda i: (i, 0, 0)),
            pl.BlockSpec((c1, 9 * cin), lambda i: (0, 0)),
            pl.BlockSpec((c1, 1), lambda i: (0, 0)),
            pl.BlockSpec((1, L1), lambda i: (0, 0)),
        ],
        out_specs=pl.BlockSpec((1, c1, 2), lambda i: (i, 0, 0)),
        out_shape=jax.ShapeDtypeStruct((n, c1, 2), jnp.float32),
        scratch_shapes=[pltpu.VMEM((9 * cin, L1), jnp.bfloat16)],
        compiler_params=_cparams(),
        cost_estimate=pl.CostEstimate(
            flops=2 * n * L1 * 9 * cin * c1, transcendentals=0,
            bytes_accessed=2 * n * cin * L1 + 4 * n * c1 * 2),
    )(xpf, w1m.astype(jnp.bfloat16), b1.reshape(-1, 1), mask1)

    sc1, sh1 = _finalize_bn(st1, g1, beta1, n * h * w)
    w1f = (w1m * sc1[:, None]).astype(jnp.bfloat16)
    b1f = (sc1 * b1 + sh1).reshape(-1, 1)
    sel = jnp.zeros((w + 1, wd), jnp.float32) \
             .at[jnp.arange(1, w + 1, 2), jnp.arange(wd)].set(1.0)

    y3, st3 = pl.pallas_call(
        functools.partial(_main_kernel, off1=off1, off2=off2, h=h, w=w),
        grid=(n,),
        in_specs=[
            pl.BlockSpec((1, cin, L1), lambda i: (i, 0, 0)),
            pl.BlockSpec((c1, 9 * cin), lambda i: (0, 0)),
            pl.BlockSpec((c1, 1), lambda i: (0, 0)),
            pl.BlockSpec((c2, 9 * c1), lambda i: (0, 0)),
            pl.BlockSpec((c2, 1), lambda i: (0, 0)),
            pl.BlockSpec((c3, 9 * c2), lambda i: (0, 0)),
            pl.BlockSpec((c3, 1), lambda i: (0, 0)),
            pl.BlockSpec((1, L1), lambda i: (0, 0)),
            pl.BlockSpec((1, L2), lambda i: (0, 0)),
            pl.BlockSpec((w + 1, wd), lambda i: (0, 0)),
        ],
        out_specs=(
            pl.BlockSpec((1, c3, L2), lambda i: (i, 0, 0)),
            pl.BlockSpec((1, c3, 2), lambda i: (i, 0, 0)),
        ),
        out_shape=(
            jax.ShapeDtypeStruct((n, c3, L2), jnp.bfloat16),
            jax.ShapeDtypeStruct((n, c3, 2), jnp.float32),
        ),
        scratch_shapes=[
            pltpu.VMEM((9 * cin, L1), jnp.bfloat16),
            pltpu.VMEM((c1, h + 2, w + 2), jnp.float32),
            pltpu.VMEM((c1 * (h + 2) + 2, 128), jnp.float32),
            pltpu.VMEM((c1, hd + 2, wd + 2), jnp.bfloat16),
            pltpu.VMEM((9 * c1, L2), jnp.bfloat16),
            pltpu.VMEM((9 * c2, L2), jnp.bfloat16),
        ],
        compiler_params=_cparams(),
        cost_estimate=pl.CostEstimate(
            flops=2 * n * (L1 * 9 * cin * c1 + L2 * 9 * (c1 * c2 + c2 * c3)),
            transcendentals=0,
            bytes_accessed=2 * n * (cin * L1 + c3 * L2) + 4 * n * c3 * 2),
    )(xpf, w1f, b1f,
      _conv_weight_mat(w2).astype(jnp.bfloat16), b2.reshape(-1, 1),
      _conv_weight_mat(w3).astype(jnp.bfloat16), b3.reshape(-1, 1),
      mask1, mask2, sel)

    sc3, sh3 = _finalize_bn(st3, g3, beta3, n * hd * wd)
    wf_flat = jnp.pad(wf.reshape(c3, hd, wd),
                      ((0, 0), (1, 1), (1, 1))).reshape(c3, L2)

    part = pl.pallas_call(
        _head_kernel,
        grid=(n,),
        in_specs=[
            pl.BlockSpec((1, c3, L2), lambda i: (i, 0, 0)),
            pl.BlockSpec((c3, 1), lambda i: (0, 0)),
            pl.BlockSpec((c3, 1), lambda i: (0, 0)),
            pl.BlockSpec((c3, L2), lambda i: (0, 0)),
        ],
        out_specs=pl.BlockSpec((1, c3, 1), lambda i: (i, 0, 0)),
        out_shape=jax.ShapeDtypeStruct((n, c3, 1), jnp.float32),
        compiler_params=_cparams(),
        cost_estimate=pl.CostEstimate(
            flops=4 * n * c3 * L2, transcendentals=0,
            bytes_accessed=2 * n * c3 * L2 + 4 * (c3 * L2 + n * c3)),
    )(y3, sc3.reshape(-1, 1), sh3.reshape(-1, 1), wf_flat)

    return jnp.sum(part[:, :, 0], axis=1, keepdims=True) + bf.reshape(1, 1)


# speedup vs baseline: 2.3496x; 1.0463x over previous
import functools

import jax
import jax.numpy as jnp
from jax import lax
from jax.experimental import pallas as pl
from jax.experimental.pallas import tpu as pltpu

EPS = 1e-5
VMEM_LIMIT = 32 * 1024 * 1024


def _cparams():
    return pltpu.CompilerParams(dimension_semantics=("parallel",),
                                vmem_limit_bytes=VMEM_LIMIT)


def _tap_offsets(wp):
    return tuple((dy - 1) * wp + (dx - 1) for dy in range(3) for dx in range(3))


def _conv_weight_mat(w_oihw, slab=None):
    o, i, kh, kw = w_oihw.shape
    m = jnp.transpose(w_oihw, (0, 2, 3, 1))
    if slab is not None and slab > i:
        m = jnp.pad(m, ((0, 0), (0, 0), (0, 0), (0, slab - i)))
        i = slab
    return m.reshape(o, kh * kw * i)


def _interior_mask(h, w):
    m = jnp.zeros((h + 2, w + 2), jnp.float32)
    m = m.at[1:h + 1, 1:w + 1].set(1.0)
    return m.reshape(1, (h + 2) * (w + 2))


def _build_patches(x, p_ref, offsets):
    cin, L = x.shape
    for t, off in enumerate(offsets):
        r = t * cin
        if off >= 0:
            p_ref[r:r + cin, 0:L - off] = x[:, off:L]
        else:
            p_ref[r:r + cin, -off:L] = x[:, 0:L + off]


def _stats1_kernel(x_ref, w_ref, b_ref, m_ref, st_ref, p_ref, *, offsets):
    _build_patches(x_ref[0], p_ref, offsets)
    y = jnp.dot(w_ref[...], p_ref[...], preferred_element_type=jnp.float32)
    y = y + b_ref[...]
    ym = jnp.where(m_ref[...] > 0.5, y, 0.0)
    st_ref[0, :, 0:1] = jnp.sum(ym, axis=1, keepdims=True)
    st_ref[0, :, 1:2] = jnp.sum(ym * ym, axis=1, keepdims=True)


def _main_kernel(x_ref, w1_ref, b1_ref, w2_ref, b2_ref, w3_ref, b3_ref,
                 m1_ref, m2_ref, s_ref, y3_ref, st_ref,
                 p1_ref, f_ref, d_ref, q_ref, p2_ref, p3_ref,
                 *, off1, off2, h, w):
    hd, wd = h // 2, w // 2
    c1 = w1_ref.shape[0]
    R = c1 * (h + 2)
    _build_patches(x_ref[0], p1_ref, off1)
    y1 = jnp.dot(w1_ref[...], p1_ref[...], preferred_element_type=jnp.float32)
    a1 = jnp.where(m1_ref[...] > 0.5,
                   jnp.maximum(y1 + b1_ref[...], 0.0),
                   0.0).astype(jnp.bfloat16)
    f_ref[...] = a1.reshape(c1, h + 2, w + 2)
    A = f_ref[...].reshape(R, w + 2)
    hm = jnp.maximum(A[:, 0:w + 1], A[:, 1:w + 2])
    dec = jnp.dot(hm, s_ref[...], preferred_element_type=jnp.float32)
    d_ref[0:R, 0:wd] = dec
    v0 = d_ref[1:R + 1:2, 0:wd]
    v1 = d_ref[2:R + 2:2, 0:wd]
    pooled = jnp.maximum(v0, v1).astype(jnp.bfloat16)
    q_ref[...] = jnp.zeros(q_ref.shape, q_ref.dtype)
    for c in range(c1):
        q_ref[c, 1:hd + 1, 1:wd + 1] = pooled[c * (hd + 1):c * (hd + 1) + hd, :]
    p1 = q_ref[...].reshape(c1, (hd + 2) * (wd + 2))

    _build_patches(p1, p2_ref, off2)
    y2 = jnp.dot(w2_ref[...], p2_ref[...], preferred_element_type=jnp.float32)
    interior = m2_ref[...] > 0.5
    a2 = jnp.where(interior, jnp.maximum(y2 + b2_ref[...], 0.0),
                   0.0).astype(jnp.bfloat16)

    _build_patches(a2, p3_ref, off2)
    y3 = jnp.dot(w3_ref[...], p3_ref[...], preferred_element_type=jnp.float32)
    y3 = jnp.where(interior, y3 + b3_ref[...], 0.0)
    st_ref[0, :, 0:1] = jnp.sum(y3, axis=1, keepdims=True)
    st_ref[0, :, 1:2] = jnp.sum(y3 * y3, axis=1, keepdims=True)
    y3_ref[0] = y3.astype(jnp.bfloat16)


def _head_kernel(y_ref, sc_ref, sh_ref, wf_ref, o_ref):
    y = y_ref[0].astype(jnp.float32)
    a = jnp.maximum(y * sc_ref[...] + sh_ref[...], 0.0)
    o_ref[0] = jnp.sum(a * wf_ref[...], axis=1, keepdims=True)


def _finalize_bn(stats, gamma, beta, count):
    tot = jnp.sum(stats, axis=0)
    mean = tot[:, 0] / count
    var = jnp.maximum(tot[:, 1] / count - mean * mean, 0.0)
    scale = gamma * lax.rsqrt(var + EPS)
    shift = beta - mean * scale
    return scale, shift


def kernel(x_nchw, w1, b1, g1, beta1, w2, b2, w3, b3, g3, beta3, wf, bf):
    n, cin, h, w = x_nchw.shape
    c1, c2, c3 = w1.shape[0], w2.shape[0], w3.shape[0]
    hd, wd = h // 2, w // 2
    L1 = (h + 2) * (w + 2)
    L2 = (hd + 2) * (wd + 2)
    off1 = _tap_offsets(w + 2)
    off2 = _tap_offsets(wd + 2)

    cp = -(-cin // 4) * 4
    xpf = jnp.pad(x_nchw, ((0, 0), (0, cp - cin), (1, 1), (1, 1))) \
             .reshape(n, cp, L1).astype(jnp.bfloat16)
    mask1 = _interior_mask(h, w)
    mask2 = _interior_mask(hd, wd)

    w1m = _conv_weight_mat(w1)

    st1 = pl.pallas_call(
        functools.partial(_stats1_kernel, offsets=off1),
        grid=(n,),
        in_specs=[
            pl.BlockSpec((1, cp, L1), lambda i: (i, 0, 0)),
            pl.BlockSpec((c1, 9 * cp), lambda i: (0, 0)),
            pl.BlockSpec((c1, 1), lambda i: (0, 0)),
            pl.BlockSpec((1, L1), lambda i: (0, 0)),
        ],
        out_specs=pl.BlockSpec((1, c1, 2), lambda i: (i, 0, 0)),
        out_shape=jax.ShapeDtypeStruct((n, c1, 2), jnp.float32),
        scratch_shapes=[pltpu.VMEM((9 * cp, L1), jnp.bfloat16)],
        compiler_params=_cparams(),
        cost_estimate=pl.CostEstimate(
            flops=2 * n * L1 * 9 * cin * c1, transcendentals=0,
            bytes_accessed=2 * n * cin * L1 + 4 * n * c1 * 2),
    )(xpf, _conv_weight_mat(w1, slab=cp).astype(jnp.bfloat16),
      b1.reshape(-1, 1), mask1)

    sc1, sh1 = _finalize_bn(st1, g1, beta1, n * h * w)
    w1f = (_conv_weight_mat(w1, slab=cp)
           * sc1[:, None]).astype(jnp.bfloat16)
    b1f = (sc1 * b1 + sh1).reshape(-1, 1)
    sel = jnp.zeros((w + 1, wd), jnp.bfloat16) \
             .at[jnp.arange(1, w + 1, 2), jnp.arange(wd)].set(1.0)

    y3, st3 = pl.pallas_call(
        functools.partial(_main_kernel, off1=off1, off2=off2, h=h, w=w),
        grid=(n,),
        in_specs=[
            pl.BlockSpec((1, cp, L1), lambda i: (i, 0, 0)),
            pl.BlockSpec((c1, 9 * cp), lambda i: (0, 0)),
            pl.BlockSpec((c1, 1), lambda i: (0, 0)),
            pl.BlockSpec((c2, 9 * c1), lambda i: (0, 0)),
            pl.BlockSpec((c2, 1), lambda i: (0, 0)),
            pl.BlockSpec((c3, 9 * c2), lambda i: (0, 0)),
            pl.BlockSpec((c3, 1), lambda i: (0, 0)),
            pl.BlockSpec((1, L1), lambda i: (0, 0)),
            pl.BlockSpec((1, L2), lambda i: (0, 0)),
            pl.BlockSpec((w + 1, wd), lambda i: (0, 0)),
        ],
        out_specs=(
            pl.BlockSpec((1, c3, L2), lambda i: (i, 0, 0)),
            pl.BlockSpec((1, c3, 2), lambda i: (i, 0, 0)),
        ),
        out_shape=(
            jax.ShapeDtypeStruct((n, c3, L2), jnp.bfloat16),
            jax.ShapeDtypeStruct((n, c3, 2), jnp.float32),
        ),
        scratch_shapes=[
            pltpu.VMEM((9 * cp, L1), jnp.bfloat16),
            pltpu.VMEM((c1, h + 2, w + 2), jnp.bfloat16),
            pltpu.VMEM((c1 * (h + 2) + 2, 128), jnp.float32),
            pltpu.VMEM((c1, hd + 2, wd + 2), jnp.bfloat16),
            pltpu.VMEM((9 * c1, L2), jnp.bfloat16),
            pltpu.VMEM((9 * c2, L2), jnp.bfloat16),
        ],
        compiler_params=_cparams(),
        cost_estimate=pl.CostEstimate(
            flops=2 * n * (L1 * 9 * cin * c1 + L2 * 9 * (c1 * c2 + c2 * c3)),
            transcendentals=0,
            bytes_accessed=2 * n * (cin * L1 + c3 * L2) + 4 * n * c3 * 2),
    )(xpf, w1f, b1f,
      _conv_weight_mat(w2).astype(jnp.bfloat16),
      b2.reshape(-1, 1).astype(jnp.bfloat16),
      _conv_weight_mat(w3).astype(jnp.bfloat16), b3.reshape(-1, 1),
      mask1, mask2, sel)

    sc3, sh3 = _finalize_bn(st3, g3, beta3, n * hd * wd)
    wf_flat = jnp.pad(wf.reshape(c3, hd, wd),
                      ((0, 0), (1, 1), (1, 1))).reshape(c3, L2)

    part = pl.pallas_call(
        _head_kernel,
        grid=(n,),
        in_specs=[
            pl.BlockSpec((1, c3, L2), lambda i: (i, 0, 0)),
            pl.BlockSpec((c3, 1), lambda i: (0, 0)),
            pl.BlockSpec((c3, 1), lambda i: (0, 0)),
            pl.BlockSpec((c3, L2), lambda i: (0, 0)),
        ],
        out_specs=pl.BlockSpec((1, c3, 1), lambda i: (i, 0, 0)),
        out_shape=jax.ShapeDtypeStruct((n, c3, 1), jnp.float32),
        compiler_params=_cparams(),
        cost_estimate=pl.CostEstimate(
            flops=4 * n * c3 * L2, transcendentals=0,
            bytes_accessed=2 * n * c3 * L2 + 4 * (c3 * L2 + n * c3)),
    )(y3, sc3.reshape(-1, 1), sh3.reshape(-1, 1), wf_flat)

    return jnp.sum(part[:, :, 0], axis=1, keepdims=True) + bf.reshape(1, 1)


# speedup vs baseline: 2.4433x; 1.0399x over previous
import functools

import jax
import jax.numpy as jnp
from jax import lax
from jax.experimental import pallas as pl
from jax.experimental.pallas import tpu as pltpu

EPS = 1e-5
VMEM_LIMIT = 32 * 1024 * 1024


def _cparams():
    return pltpu.CompilerParams(dimension_semantics=("parallel",),
                                vmem_limit_bytes=VMEM_LIMIT)


def _tap_offsets(wp):
    return tuple((dy - 1) * wp + (dx - 1) for dy in range(3) for dx in range(3))


def _conv_weight_mat(w_oihw, slab=None):
    o, i, kh, kw = w_oihw.shape
    m = jnp.transpose(w_oihw, (0, 2, 3, 1))
    if slab is not None and slab > i:
        m = jnp.pad(m, ((0, 0), (0, 0), (0, 0), (0, slab - i)))
        i = slab
    return m.reshape(o, kh * kw * i)


def _interior_mask(h, w):
    m = jnp.zeros((h + 2, w + 2), jnp.float32)
    m = m.at[1:h + 1, 1:w + 1].set(1.0)
    return m.reshape(1, (h + 2) * (w + 2))


def _build_patches(x, p_ref, offsets):
    cin, L = x.shape
    for t, off in enumerate(offsets):
        r = t * cin
        if off >= 0:
            p_ref[r:r + cin, 0:L - off] = x[:, off:L]
        else:
            p_ref[r:r + cin, -off:L] = x[:, 0:L + off]


def _stats1_kernel(x_ref, w_ref, b_ref, m_ref, st_ref, p_ref, *, offsets):
    def body(g, _):
        _build_patches(x_ref[g], p_ref, offsets)
        y = jnp.dot(w_ref[...], p_ref[...], preferred_element_type=jnp.float32)
        y = y + b_ref[...]
        ym = jnp.where(m_ref[...] > 0.5, y, 0.0)
        st_ref[g, :, 0:1] = jnp.sum(ym, axis=1, keepdims=True)
        st_ref[g, :, 1:2] = jnp.sum(ym * ym, axis=1, keepdims=True)
        return 0
    lax.fori_loop(0, x_ref.shape[0], body, 0, unroll=False)


def _main_kernel(x_ref, w1_ref, b1_ref, w2_ref, b2_ref, w3_ref, b3_ref,
                 m1_ref, m2_ref, s_ref, y3_ref, st_ref,
                 p1_ref, f_ref, d_ref, q_ref, p2_ref, p3_ref,
                 *, off1, off2, h, w):
    hd, wd = h // 2, w // 2
    c1 = w1_ref.shape[0]
    R = c1 * (h + 2)
    def body(g, _):
        _build_patches(x_ref[g], p1_ref, off1)
        y1 = jnp.dot(w1_ref[...], p1_ref[...],
                     preferred_element_type=jnp.float32)
        a1 = jnp.where(m1_ref[...] > 0.5,
                       jnp.maximum(y1 + b1_ref[...], 0.0),
                       0.0).astype(jnp.bfloat16)
        f_ref[...] = a1.reshape(c1, h + 2, w + 2)
        A = f_ref[...].reshape(R, w + 2)
        hm = jnp.maximum(A[:, 0:w + 1], A[:, 1:w + 2])
        dec = jnp.dot(hm, s_ref[...], preferred_element_type=jnp.float32)
        d_ref[0:R, 0:wd] = dec
        v0 = d_ref[1:R + 1:2, 0:wd]
        v1 = d_ref[2:R + 2:2, 0:wd]
        pooled = jnp.maximum(v0, v1).astype(jnp.bfloat16)
        q_ref[...] = jnp.zeros(q_ref.shape, q_ref.dtype)
        for c in range(c1):
            q_ref[c, 1:hd + 1, 1:wd + 1] = \
                pooled[c * (hd + 1):c * (hd + 1) + hd, :]
        p1 = q_ref[...].reshape(c1, (hd + 2) * (wd + 2))

        _build_patches(p1, p2_ref, off2)
        y2 = jnp.dot(w2_ref[...], p2_ref[...],
                     preferred_element_type=jnp.float32)
        interior = m2_ref[...] > 0.5
        a2 = jnp.where(interior, jnp.maximum(y2 + b2_ref[...], 0.0),
                       0.0).astype(jnp.bfloat16)

        _build_patches(a2, p3_ref, off2)
        y3 = jnp.dot(w3_ref[...], p3_ref[...],
                     preferred_element_type=jnp.float32)
        y3 = jnp.where(interior, y3 + b3_ref[...], 0.0)
        st_ref[g, :, 0:1] = jnp.sum(y3, axis=1, keepdims=True)
        st_ref[g, :, 1:2] = jnp.sum(y3 * y3, axis=1, keepdims=True)
        y3_ref[g] = y3.astype(jnp.bfloat16)
        return 0
    lax.fori_loop(0, x_ref.shape[0], body, 0, unroll=False)


def _head_kernel(y_ref, sc_ref, sh_ref, wf_ref, o_ref):
    def body(g, _):
        y = y_ref[g].astype(jnp.float32)
        a = jnp.maximum(y * sc_ref[...] + sh_ref[...], 0.0)
        o_ref[g] = jnp.sum(a * wf_ref[...], axis=1, keepdims=True)
        return 0
    lax.fori_loop(0, y_ref.shape[0], body, 0, unroll=False)


def _finalize_bn(stats, gamma, beta, count):
    tot = jnp.sum(stats, axis=0)
    mean = tot[:, 0] / count
    var = jnp.maximum(tot[:, 1] / count - mean * mean, 0.0)
    scale = gamma * lax.rsqrt(var + EPS)
    shift = beta - mean * scale
    return scale, shift


def kernel(x_nchw, w1, b1, g1, beta1, w2, b2, w3, b3, g3, beta3, wf, bf):
    n, cin, h, w = x_nchw.shape
    c1, c2, c3 = w1.shape[0], w2.shape[0], w3.shape[0]
    hd, wd = h // 2, w // 2
    L1 = (h + 2) * (w + 2)
    L2 = (hd + 2) * (wd + 2)
    off1 = _tap_offsets(w + 2)
    off2 = _tap_offsets(wd + 2)

    G = next(g for g in (8, 4, 2, 1) if n % g == 0)
    cp = -(-cin // 4) * 4
    xpf = jnp.pad(x_nchw, ((0, 0), (0, cp - cin), (1, 1), (1, 1))) \
             .reshape(n, cp, L1).astype(jnp.bfloat16)
    mask1 = _interior_mask(h, w)
    mask2 = _interior_mask(hd, wd)

    w1m = _conv_weight_mat(w1)

    st1 = pl.pallas_call(
        functools.partial(_stats1_kernel, offsets=off1),
        grid=(n // G,),
        in_specs=[
            pl.BlockSpec((G, cp, L1), lambda i: (i, 0, 0)),
            pl.BlockSpec((c1, 9 * cp), lambda i: (0, 0)),
            pl.BlockSpec((c1, 1), lambda i: (0, 0)),
            pl.BlockSpec((1, L1), lambda i: (0, 0)),
        ],
        out_specs=pl.BlockSpec((G, c1, 2), lambda i: (i, 0, 0)),
        out_shape=jax.ShapeDtypeStruct((n, c1, 2), jnp.float32),
        scratch_shapes=[pltpu.VMEM((9 * cp, L1), jnp.bfloat16)],
        compiler_params=_cparams(),
        cost_estimate=pl.CostEstimate(
            flops=2 * n * L1 * 9 * cin * c1, transcendentals=0,
            bytes_accessed=2 * n * cin * L1 + 4 * n * c1 * 2),
    )(xpf, _conv_weight_mat(w1, slab=cp).astype(jnp.bfloat16),
      b1.reshape(-1, 1), mask1)

    sc1, sh1 = _finalize_bn(st1, g1, beta1, n * h * w)
    w1f = (_conv_weight_mat(w1, slab=cp)
           * sc1[:, None]).astype(jnp.bfloat16)
    b1f = (sc1 * b1 + sh1).reshape(-1, 1)
    sel = jnp.zeros((w + 1, wd), jnp.bfloat16) \
             .at[jnp.arange(1, w + 1, 2), jnp.arange(wd)].set(1.0)

    y3, st3 = pl.pallas_call(
        functools.partial(_main_kernel, off1=off1, off2=off2, h=h, w=w),
        grid=(n // G,),
        in_specs=[
            pl.BlockSpec((G, cp, L1), lambda i: (i, 0, 0)),
            pl.BlockSpec((c1, 9 * cp), lambda i: (0, 0)),
            pl.BlockSpec((c1, 1), lambda i: (0, 0)),
            pl.BlockSpec((c2, 9 * c1), lambda i: (0, 0)),
            pl.BlockSpec((c2, 1), lambda i: (0, 0)),
            pl.BlockSpec((c3, 9 * c2), lambda i: (0, 0)),
            pl.BlockSpec((c3, 1), lambda i: (0, 0)),
            pl.BlockSpec((1, L1), lambda i: (0, 0)),
            pl.BlockSpec((1, L2), lambda i: (0, 0)),
            pl.BlockSpec((w + 1, wd), lambda i: (0, 0)),
        ],
        out_specs=(
            pl.BlockSpec((G, c3, L2), lambda i: (i, 0, 0)),
            pl.BlockSpec((G, c3, 2), lambda i: (i, 0, 0)),
        ),
        out_shape=(
            jax.ShapeDtypeStruct((n, c3, L2), jnp.bfloat16),
            jax.ShapeDtypeStruct((n, c3, 2), jnp.float32),
        ),
        scratch_shapes=[
            pltpu.VMEM((9 * cp, L1), jnp.bfloat16),
            pltpu.VMEM((c1, h + 2, w + 2), jnp.bfloat16),
            pltpu.VMEM((c1 * (h + 2) + 2, 128), jnp.float32),
            pltpu.VMEM((c1, hd + 2, wd + 2), jnp.bfloat16),
            pltpu.VMEM((9 * c1, L2), jnp.bfloat16),
            pltpu.VMEM((9 * c2, L2), jnp.bfloat16),
        ],
        compiler_params=_cparams(),
        cost_estimate=pl.CostEstimate(
            flops=2 * n * (L1 * 9 * cin * c1 + L2 * 9 * (c1 * c2 + c2 * c3)),
            transcendentals=0,
            bytes_accessed=2 * n * (cin * L1 + c3 * L2) + 4 * n * c3 * 2),
    )(xpf, w1f, b1f,
      _conv_weight_mat(w2).astype(jnp.bfloat16),
      b2.reshape(-1, 1).astype(jnp.bfloat16),
      _conv_weight_mat(w3).astype(jnp.bfloat16), b3.reshape(-1, 1),
      mask1, mask2, sel)

    sc3, sh3 = _finalize_bn(st3, g3, beta3, n * hd * wd)
    wf_flat = jnp.pad(wf.reshape(c3, hd, wd),
                      ((0, 0), (1, 1), (1, 1))).reshape(c3, L2)

    part = pl.pallas_call(
        _head_kernel,
        grid=(n // G,),
        in_specs=[
            pl.BlockSpec((G, c3, L2), lambda i: (i, 0, 0)),
            pl.BlockSpec((c3, 1), lambda i: (0, 0)),
            pl.BlockSpec((c3, 1), lambda i: (0, 0)),
            pl.BlockSpec((c3, L2), lambda i: (0, 0)),
        ],
        out_specs=pl.BlockSpec((G, c3, 1), lambda i: (i, 0, 0)),
        out_shape=jax.ShapeDtypeStruct((n, c3, 1), jnp.float32),
        compiler_params=_cparams(),
        cost_estimate=pl.CostEstimate(
            flops=4 * n * c3 * L2, transcendentals=0,
            bytes_accessed=2 * n * c3 * L2 + 4 * (c3 * L2 + n * c3)),
    )(y3, sc3.reshape(-1, 1), sh3.reshape(-1, 1), wf_flat)

    return jnp.sum(part[:, :, 0], axis=1, keepdims=True) + bf.reshape(1, 1)
